# Initial kernel scaffold; baseline (speedup 1.0000x reference)
import jax
import jax.numpy as jnp
from jax.experimental import pallas as pl


def kernel(res1, res2, res3, res4, pc1_w, pc1_g, pc1_b, pc1_m, pc1_v, p3_g1, p3_g2, p3_sp0_w, p3_sp0_b, p3_sp1_w, p3_sp1_b, p3_cx0_w, p3_cx0_b, p3_cx1_w, p3_cx1_b, p3_sa_cbr_w, p3_sa_cbr_g, p3_sa_cbr_b, p3_sa_cbr_m, p3_sa_cbr_v, p3_sa_c11_w, p3_sa_c11_g, p3_sa_c11_b, p3_sa_c11_m, p3_sa_c11_v, p3_ca_cbr_w, p3_ca_cbr_g, p3_ca_cbr_b, p3_ca_cbr_m, p3_ca_cbr_v, p3_ca_c11_w, p3_ca_c11_g, p3_ca_c11_b, p3_ca_c11_m, p3_ca_c11_v, p3_ch_w, p3_ch_g, p3_ch_b, p3_ch_m, p3_ch_v, p3_sm_w, p3_sm_g, p3_sm_b, p3_sm_m, p3_sm_v, p2_g1, p2_g2, p2_sp0_w, p2_sp0_b, p2_sp1_w, p2_sp1_b, p2_cx0_w, p2_cx0_b, p2_cx1_w, p2_cx1_b, p2_sa_cbr_w, p2_sa_cbr_g, p2_sa_cbr_b, p2_sa_cbr_m, p2_sa_cbr_v, p2_sa_c11_w, p2_sa_c11_g, p2_sa_c11_b, p2_sa_c11_m, p2_sa_c11_v, p2_ca_cbr_w, p2_ca_cbr_g, p2_ca_cbr_b, p2_ca_cbr_m, p2_ca_cbr_v, p2_ca_c11_w, p2_ca_c11_g, p2_ca_c11_b, p2_ca_c11_m, p2_ca_c11_v, p2_ch_w, p2_ch_g, p2_ch_b, p2_ch_m, p2_ch_v, p2_sm_w, p2_sm_g, p2_sm_b, p2_sm_m, p2_sm_v, p1_pre_w, p1_weights, p1_post_w, p1_post_g, p1_post_b, p1_post_m, p1_post_v, p1_pa_w, p1_pa_b, p1_ca1_w, p1_ca2_w, p1_sc_w, p1_sc_g, p1_sc_b, p1_sc_m, p1_sc_v, p1_proj_w, p1_proj_g, p1_proj_b, p1_proj_m, p1_proj_v, seg1_w, seg1_g, seg1_b, seg1_m, seg1_v, seg2_w):
    raise NotImplementedError("write your pallas kernel here")


# speedup vs baseline: 1.5840x; 1.5840x over previous
import functools

import jax
import jax.numpy as jnp
from jax.experimental import pallas as pl
from jax.experimental.pallas import tpu as pltpu

f32 = jnp.float32
bf16 = jnp.bfloat16


def _pad_hw(x):
    H, W, C = x.shape
    zr = jnp.zeros((1, W, C), x.dtype)
    x = jnp.concatenate([zr, x, zr], axis=0)
    zc = jnp.zeros((H + 2, 1, C), x.dtype)
    return jnp.concatenate([zc, x, zc], axis=1)


def _conv3x3(xin, w3_ref, sb):
    H, W, C = xin.shape
    xp = _pad_hw(xin)
    cols = jnp.concatenate([xp[:, 0:W], xp[:, 1:1 + W], xp[:, 2:2 + W]],
                           axis=-1)
    acc = jnp.dot(cols[0:H].reshape(H * W, 3 * C), w3_ref[0],
                  preferred_element_type=f32)
    for dy in (1, 2):
        acc = acc + jnp.dot(cols[dy:dy + H].reshape(H * W, 3 * C), w3_ref[dy],
                            preferred_element_type=f32)
    return acc * sb[0] + sb[1]


def _up2x(x):
    H, W, C = x.shape
    pv = jnp.concatenate([x[:1], x[:-1]], axis=0)
    nx = jnp.concatenate([x[1:], x[-1:]], axis=0)
    ev = 0.25 * pv + 0.75 * x
    od = 0.75 * x + 0.25 * nx
    x = jnp.stack([ev, od], axis=1).reshape(2 * H, W, C)
    pv = jnp.concatenate([x[:, :1], x[:, :-1]], axis=1)
    nx = jnp.concatenate([x[:, 1:], x[:, -1:]], axis=1)
    ev = 0.25 * pv + 0.75 * x
    od = 0.75 * x + 0.25 * nx
    return jnp.stack([ev, od], axis=2).reshape(2 * H, 2 * W, C)


def _relu6(y):
    return jnp.clip(y, 0.0, 6.0)


def _mm_gate_kernel(a_ref, w_ref, sb_ref, o_ref):
    y = jnp.dot(a_ref[...], w_ref[...], preferred_element_type=f32)
    sb = sb_ref[...]
    y = y * sb[0] + sb[1]
    y = y * y + y
    o_ref[...] = y.astype(o_ref.dtype)


def _att2_kernel(xs_ref, xc_ref, ws_ref, sbs_ref, wc_ref, sbc_ref,
                 fs_ref, ps_ref, fc_ref, pc_ref):
    for x_ref, w_ref, sb_ref, f_ref, p_ref in (
            (xs_ref, ws_ref, sbs_ref, fs_ref, ps_ref),
            (xc_ref, wc_ref, sbc_ref, fc_ref, pc_ref)):
        y = _relu6(_conv3x3(x_ref[0], w_ref, sb_ref[...]))
        fb = y.astype(bf16)
        f_ref[0] = fb
        pooled = jnp.mean(fb.astype(f32), axis=0, keepdims=True)
        p_ref[0] = jnp.broadcast_to(pooled, (8, pooled.shape[1]))


def _csim_tail_kernel(c_ref, s_ref, rc_ref, rs_ref, wch_ref, sbch_ref,
                      wsm_ref, sbsm_ref, o_ref):
    up = _up2x(c_ref[0])
    H, W, C = up.shape
    xin = up * rc_ref[0]
    ch = _relu6(_conv3x3(xin, wch_ref, sbch_ref[...])).astype(bf16)
    xin2 = s_ref[0] * rs_ref[0] + ch.reshape(H, W, C)
    y = _conv3x3(xin2, wsm_ref, sbsm_ref[...])
    y = y * y + y
    o_ref[0] = y.astype(o_ref.dtype)


def _csim1_front_kernel(r_ref, x_ref, wpre_ref, sbpre_ref, wpost_ref,
                        sbpost_ref, xp_ref, p_ref):
    r = r_ref[0]
    Hr, Wr, Ci = r.shape
    sbp = sbpre_ref[...]
    pre = jnp.dot(r.reshape(Hr * Wr, Ci), wpre_ref[...],
                  preferred_element_type=f32)
    pre = (pre * sbp[0]).astype(bf16)
    fw1 = sbp[2].astype(bf16)
    up = _up2x(x_ref[0]) * fw1
    xin = pre.reshape(up.shape) + up
    y = _relu6(_conv3x3(xin, wpost_ref, sbpost_ref[...]))
    fb = y.astype(bf16)
    xp_ref[0] = fb
    pooled = jnp.mean(fb.astype(f32), axis=0, keepdims=True)
    p_ref[0] = jnp.broadcast_to(pooled, (8, pooled.shape[1]))


def _tail_seg_kernel(xp_ref, ca_ref, paw_ref, sbpa_ref, wproj_ref, sbproj_ref,
                     wsc_ref, sbsc_ref, wseg_ref, sbseg_ref, wcls_ref, o_ref,
                     *, H, W):
    C = xp_ref.shape[-1]
    xpv = xp_ref[0].reshape(H, W, C)
    xq = _pad_hw(xpv)
    acc = None
    for t in range(9):
        dy, dx = divmod(t, 3)
        p = xq[dy:dy + H, dx:dx + W, :].astype(f32)
        term = p * paw_ref[t]
        acc = term if acc is None else acc + term
    sbpa = sbpa_ref[...]
    y = acc.reshape(H * W, C) * sbpa[0] + sbpa[1]
    y = jax.nn.sigmoid(y)
    ctr = xpv.reshape(H * W, C).astype(f32)
    gated = ((y + ca_ref[0]) * ctr).astype(bf16)

    yp = _conv3x3(gated.reshape(H, W, C), wproj_ref, sbproj_ref[...])
    sc = jnp.dot(xpv.reshape(H * W, C), wsc_ref[...],
                 preferred_element_type=f32)
    sbsc = sbsc_ref[...]
    x64 = _relu6(yp + sc * sbsc[0] + sbsc[1]).astype(bf16)

    up = _up2x(x64.reshape(H, W, C))
    ys = _relu6(_conv3x3(up, wseg_ref, sbseg_ref[...]))
    o_ref[0] = jnp.dot(ys.astype(bf16), wcls_ref[...],
                       preferred_element_type=f32)


def _bn_fold(g, b, m, v):
    s = g / jnp.sqrt(v + 1e-5)
    return s, b - m * s


def _sb_pack(c, scale=None, bias=None, extra=None):
    sb = jnp.zeros((8, c), f32)
    sb = sb.at[0].set(jnp.ones((c,), f32) if scale is None else scale)
    if bias is not None:
        sb = sb.at[1].set(bias)
    if extra is not None:
        sb = sb.at[2].set(extra)
    return sb


def _w3(w):
    return w.reshape(3, 3 * w.shape[2], w.shape[3]).astype(bf16)


_PPAR = pltpu.CompilerParams(dimension_semantics=("parallel",))


def _full(shape):
    nd = len(shape)
    return pl.BlockSpec(shape, lambda n, _nd=nd: (0,) * _nd)


def _img(shape):
    nd = len(shape) - 1
    return pl.BlockSpec((1,) + shape[1:],
                        lambda n, _nd=nd: (n,) + (0,) * _nd)


def _att2(xs, ws, sbs, xc, wc, sbc):
    N, Hs, Ws, Cs = xs.shape
    _, Hc, Wc, Cc = xc.shape
    Co = ws.shape[-1]
    fs, ps, fc, pc = pl.pallas_call(
        _att2_kernel,
        out_shape=(jax.ShapeDtypeStruct((N, Hs * Ws, Co), bf16),
                   jax.ShapeDtypeStruct((N, 8, Co), f32),
                   jax.ShapeDtypeStruct((N, Hc * Wc, Co), bf16),
                   jax.ShapeDtypeStruct((N, 8, Co), f32)),
        grid=(N,),
        in_specs=[_img(xs.shape), _img(xc.shape),
                  _full(ws.shape), _full(sbs.shape),
                  _full(wc.shape), _full(sbc.shape)],
        out_specs=(pl.BlockSpec((1, Hs * Ws, Co), lambda n: (n, 0, 0)),
                   pl.BlockSpec((1, 8, Co), lambda n: (n, 0, 0)),
                   pl.BlockSpec((1, Hc * Wc, Co), lambda n: (n, 0, 0)),
                   pl.BlockSpec((1, 8, Co), lambda n: (n, 0, 0))),
        compiler_params=_PPAR,
    )(xs, xc, ws, sbs, wc, sbc)
    return (fs.reshape(N, Hs, Ws, Co), ps[:, 0, :],
            fc.reshape(N, Hc, Wc, Co), pc[:, 0, :])


def _att_1x1(pooled, w11, g, b, m, v):
    c = w11.shape[-1]
    s, t = _bn_fold(g, b, m, v)
    return jax.nn.sigmoid((pooled @ w11.reshape(c, c)) * s + t)


def _csim_mid(s_att, c_att, g1, g2, sp, cx):
    N, C = s_att.shape
    r = 16

    def l2n(v):
        nrm = jnp.sqrt(jnp.sum(v * v, axis=2, keepdims=True))
        return v / jnp.maximum(nrm, 1e-12)

    ss = l2n(s_att.reshape(N, r, C // r))
    cs = l2n(c_att.reshape(N, r, C // r))
    aff = jnp.einsum("brk,bsk->brs", ss, cs).reshape(N, r * r)

    def mlp(w0, b0, w1, b1):
        h = jnp.maximum(aff @ w0 + b0, 0.0)
        return jnp.maximum(h @ w1 + b1, 0.0)

    re_s = jax.nn.sigmoid(s_att + g1 * mlp(*sp))
    re_c = jax.nn.sigmoid(c_att + g2 * mlp(*cx))
    return re_s, re_c


def _csim_tail(c_feat, s_feat, re_c, re_s, wch, sbch, wsm, sbsm):
    N, H, W, C = s_feat.shape
    rc = re_c.reshape(N, 1, C).astype(bf16)
    rs = re_s.reshape(N, 1, C).astype(bf16)
    out = pl.pallas_call(
        _csim_tail_kernel,
        out_shape=jax.ShapeDtypeStruct((N, H * W, C), bf16),
        grid=(N,),
        in_specs=[_img(c_feat.shape), _img(s_feat.shape),
                  pl.BlockSpec((1, 1, C), lambda n: (n, 0, 0)),
                  pl.BlockSpec((1, 1, C), lambda n: (n, 0, 0)),
                  _full(wch.shape), _full(sbch.shape),
                  _full(wsm.shape), _full(sbsm.shape)],
        out_specs=pl.BlockSpec((1, H * W, C), lambda n: (n, 0, 0)),
        compiler_params=_PPAR,
    )(c_feat, s_feat, rc, rs, wch, sbch, wsm, sbsm)
    return out.reshape(N, H, W, C)


def kernel(res1, res2, res3, res4, pc1_w, pc1_g, pc1_b, pc1_m, pc1_v, p3_g1, p3_g2, p3_sp0_w, p3_sp0_b, p3_sp1_w, p3_sp1_b, p3_cx0_w, p3_cx0_b, p3_cx1_w, p3_cx1_b, p3_sa_cbr_w, p3_sa_cbr_g, p3_sa_cbr_b, p3_sa_cbr_m, p3_sa_cbr_v, p3_sa_c11_w, p3_sa_c11_g, p3_sa_c11_b, p3_sa_c11_m, p3_sa_c11_v, p3_ca_cbr_w, p3_ca_cbr_g, p3_ca_cbr_b, p3_ca_cbr_m, p3_ca_cbr_v, p3_ca_c11_w, p3_ca_c11_g, p3_ca_c11_b, p3_ca_c11_m, p3_ca_c11_v, p3_ch_w, p3_ch_g, p3_ch_b, p3_ch_m, p3_ch_v, p3_sm_w, p3_sm_g, p3_sm_b, p3_sm_m, p3_sm_v, p2_g1, p2_g2, p2_sp0_w, p2_sp0_b, p2_sp1_w, p2_sp1_b, p2_cx0_w, p2_cx0_b, p2_cx1_w, p2_cx1_b, p2_sa_cbr_w, p2_sa_cbr_g, p2_sa_cbr_b, p2_sa_cbr_m, p2_sa_cbr_v, p2_sa_c11_w, p2_sa_c11_g, p2_sa_c11_b, p2_sa_c11_m, p2_sa_c11_v, p2_ca_cbr_w, p2_ca_cbr_g, p2_ca_cbr_b, p2_ca_cbr_m, p2_ca_cbr_v, p2_ca_c11_w, p2_ca_c11_g, p2_ca_c11_b, p2_ca_c11_m, p2_ca_c11_v, p2_ch_w, p2_ch_g, p2_ch_b, p2_ch_m, p2_ch_v, p2_sm_w, p2_sm_g, p2_sm_b, p2_sm_m, p2_sm_v, p1_pre_w, p1_weights, p1_post_w, p1_post_g, p1_post_b, p1_post_m, p1_post_v, p1_pa_w, p1_pa_b, p1_ca1_w, p1_ca2_w, p1_sc_w, p1_sc_g, p1_sc_b, p1_sc_m, p1_sc_v, p1_proj_w, p1_proj_g, p1_proj_b, p1_proj_m, p1_proj_v, seg1_w, seg1_g, seg1_b, seg1_m, seg1_v, seg2_w):
    t = lambda a: jnp.transpose(a, (0, 2, 3, 1)).astype(bf16)
    r1, r2, r3, r4 = t(res1), t(res2), t(res3), t(res4)
    N, H4, W4, C4 = r4.shape
    dc = 32

    s, b = _bn_fold(pc1_g, pc1_b, pc1_m, pc1_v)
    M = N * H4 * W4
    x8 = pl.pallas_call(
        _mm_gate_kernel,
        out_shape=jax.ShapeDtypeStruct((M, dc), bf16),
        grid=(4,),
        in_specs=[pl.BlockSpec((M // 4, C4), lambda i: (i, 0)),
                  pl.BlockSpec((C4, dc), lambda i: (0, 0)),
                  pl.BlockSpec((8, dc), lambda i: (0, 0))],
        out_specs=pl.BlockSpec((M // 4, dc), lambda i: (i, 0)),
        compiler_params=_PPAR,
    )(r4.reshape(M, C4).astype(bf16),
      pc1_w.reshape(C4, dc).astype(bf16), _sb_pack(dc, s, b))
    x = x8.reshape(N, H4, W4, dc)

    for (sp_feat, g1, g2, sp0_w, sp0_b, sp1_w, sp1_b, cx0_w, cx0_b, cx1_w,
         cx1_b, sa_cbr_w, sa_cbr_g, sa_cbr_b, sa_cbr_m, sa_cbr_v, sa_c11_w,
         sa_c11_g, sa_c11_b, sa_c11_m, sa_c11_v, ca_cbr_w, ca_cbr_g, ca_cbr_b,
         ca_cbr_m, ca_cbr_v, ca_c11_w, ca_c11_g, ca_c11_b, ca_c11_m, ca_c11_v,
         ch_w, ch_g, ch_b, ch_m, ch_v, sm_w, sm_g, sm_b, sm_m, sm_v) in (
            (r3, p3_g1, p3_g2, p3_sp0_w, p3_sp0_b, p3_sp1_w, p3_sp1_b,
             p3_cx0_w, p3_cx0_b, p3_cx1_w, p3_cx1_b,
             p3_sa_cbr_w, p3_sa_cbr_g, p3_sa_cbr_b, p3_sa_cbr_m, p3_sa_cbr_v,
             p3_sa_c11_w, p3_sa_c11_g, p3_sa_c11_b, p3_sa_c11_m, p3_sa_c11_v,
             p3_ca_cbr_w, p3_ca_cbr_g, p3_ca_cbr_b, p3_ca_cbr_m, p3_ca_cbr_v,
             p3_ca_c11_w, p3_ca_c11_g, p3_ca_c11_b, p3_ca_c11_m, p3_ca_c11_v,
             p3_ch_w, p3_ch_g, p3_ch_b, p3_ch_m, p3_ch_v,
             p3_sm_w, p3_sm_g, p3_sm_b, p3_sm_m, p3_sm_v),
            (r2, p2_g1, p2_g2, p2_sp0_w, p2_sp0_b, p2_sp1_w, p2_sp1_b,
             p2_cx0_w, p2_cx0_b, p2_cx1_w, p2_cx1_b,
             p2_sa_cbr_w, p2_sa_cbr_g, p2_sa_cbr_b, p2_sa_cbr_m, p2_sa_cbr_v,
             p2_sa_c11_w, p2_sa_c11_g, p2_sa_c11_b, p2_sa_c11_m, p2_sa_c11_v,
             p2_ca_cbr_w, p2_ca_cbr_g, p2_ca_cbr_b, p2_ca_cbr_m, p2_ca_cbr_v,
             p2_ca_c11_w, p2_ca_c11_g, p2_ca_c11_b, p2_ca_c11_m, p2_ca_c11_v,
             p2_ch_w, p2_ch_g, p2_ch_b, p2_ch_m, p2_ch_v,
             p2_sm_w, p2_sm_g, p2_sm_b, p2_sm_m, p2_sm_v)):
        ss, st = _bn_fold(sa_cbr_g, sa_cbr_b, sa_cbr_m, sa_cbr_v)
        cs, ct = _bn_fold(ca_cbr_g, ca_cbr_b, ca_cbr_m, ca_cbr_v)
        s_feat, s_pool, c_feat, c_pool = _att2(
            sp_feat, _w3(sa_cbr_w), _sb_pack(dc, ss, st),
            x, _w3(ca_cbr_w), _sb_pack(dc, cs, ct))
        s_att = _att_1x1(s_pool, sa_c11_w, sa_c11_g, sa_c11_b, sa_c11_m,
                         sa_c11_v)
        c_att = _att_1x1(c_pool, ca_c11_w, ca_c11_g, ca_c11_b, ca_c11_m,
                         ca_c11_v)
        re_s, re_c = _csim_mid(s_att, c_att, g1, g2,
                               (sp0_w, sp0_b, sp1_w, sp1_b),
                               (cx0_w, cx0_b, cx1_w, cx1_b))
        chs, cht = _bn_fold(ch_g, ch_b, ch_m, ch_v)
        sms, smt = _bn_fold(sm_g, sm_b, sm_m, sm_v)
        x = _csim_tail(c_feat, s_feat, re_c, re_s,
                       _w3(ch_w), _sb_pack(dc, chs, cht),
                       _w3(sm_w), _sb_pack(dc, sms, smt))

    N1, Hr, Wr, Ci = r1.shape
    w_pos = jnp.maximum(p1_weights, 0.0)
    fw = w_pos / (jnp.sum(w_pos) + 1e-8)
    ps, pt = _bn_fold(p1_post_g, p1_post_b, p1_post_m, p1_post_v)
    sb_pre = _sb_pack(dc, fw[0] * jnp.ones((dc,), f32), None,
                      fw[1] * jnp.ones((dc,), f32))
    xp, pooled = pl.pallas_call(
        _csim1_front_kernel,
        out_shape=(jax.ShapeDtypeStruct((N, Hr * Wr, dc), bf16),
                   jax.ShapeDtypeStruct((N, 8, dc), f32)),
        grid=(N,),
        in_specs=[_img(r1.shape), _img(x.shape),
                  _full((Ci, dc)), _full((8, dc)),
                  _full((3, 3 * dc, dc)), _full((8, dc))],
        out_specs=(pl.BlockSpec((1, Hr * Wr, dc), lambda n: (n, 0, 0)),
                   pl.BlockSpec((1, 8, dc), lambda n: (n, 0, 0))),
        compiler_params=_PPAR,
    )(r1, x, p1_pre_w.reshape(Ci, dc).astype(bf16), sb_pre,
      _w3(p1_post_w), _sb_pack(dc, ps, pt))
    pooled = pooled[:, 0, :]

    hca = jnp.clip(pooled @ p1_ca1_w.reshape(dc, dc // 16), 0.0, 6.0)
    ca = jax.nn.sigmoid(hca @ p1_ca2_w.reshape(dc // 16, dc))

    pjs, pjt = _bn_fold(p1_proj_g, p1_proj_b, p1_proj_m, p1_proj_v)
    scs, sct = _bn_fold(p1_sc_g, p1_sc_b, p1_sc_m, p1_sc_v)
    sgs, sgt = _bn_fold(seg1_g, seg1_b, seg1_m, seg1_v)
    nc = seg2_w.shape[-1]
    Ho, Wo = 2 * Hr, 2 * Wr
    out = pl.pallas_call(
        functools.partial(_tail_seg_kernel, H=Hr, W=Wr),
        out_shape=jax.ShapeDtypeStruct((N, Ho * Wo, nc), f32),
        grid=(N,),
        in_specs=[pl.BlockSpec((1, Hr * Wr, dc), lambda n: (n, 0, 0)),
                  pl.BlockSpec((1, 1, dc), lambda n: (n, 0, 0)),
                  _full((9, dc)), _full((8, dc)),
                  _full((3, 3 * dc, dc)), _full((8, dc)),
                  _full((dc, dc)), _full((8, dc)),
                  _full((3, 3 * dc, dc)), _full((8, dc)),
                  _full((dc, nc))],
        out_specs=pl.BlockSpec((1, Ho * Wo, nc), lambda n: (n, 0, 0)),
        compiler_params=_PPAR,
    )(xp, ca.reshape(N, 1, dc),
      p1_pa_w.reshape(9, dc).astype(f32), _sb_pack(dc, None, p1_pa_b),
      _w3(p1_proj_w), _sb_pack(dc, pjs, pjt),
      p1_sc_w.reshape(dc, dc).astype(bf16), _sb_pack(dc, scs, sct),
      _w3(seg1_w), _sb_pack(dc, sgs, sgt),
      seg2_w.reshape(dc, nc).astype(bf16))

    return jnp.transpose(out.reshape(N, Ho, Wo, nc), (0, 3, 1, 2))


# speedup vs baseline: 2.0577x; 1.2991x over previous
import functools

import jax
import jax.numpy as jnp
from jax.experimental import pallas as pl
from jax.experimental.pallas import tpu as pltpu

f32 = jnp.float32
bf16 = jnp.bfloat16


def _pad_hw(x):
    H, W, C = x.shape
    zr = jnp.zeros((1, W, C), x.dtype)
    x = jnp.concatenate([zr, x, zr], axis=0)
    zc = jnp.zeros((H + 2, 1, C), x.dtype)
    return jnp.concatenate([zc, x, zc], axis=1)


def _clamp_pad(x):
    x = jnp.concatenate([x[:1], x, x[-1:]], axis=0)
    return jnp.concatenate([x[:, :1], x, x[:, -1:]], axis=1)


def _conv3raw(xin, w3_ref, pad=_pad_hw):
    H, W, C = xin.shape
    xp = pad(xin)
    cols = jnp.concatenate([xp[:, 0:W], xp[:, 1:1 + W], xp[:, 2:2 + W]],
                           axis=-1)
    acc = jnp.dot(cols[0:H].reshape(H * W, 3 * C), w3_ref[0],
                  preferred_element_type=f32)
    for dy in (1, 2):
        acc = acc + jnp.dot(cols[dy:dy + H].reshape(H * W, 3 * C),
                            w3_ref[dy], preferred_element_type=f32)
    return acc


def _edge_corr(vec, wcorr):
    cp = jnp.concatenate([vec[:1], vec, vec[-1:]], axis=0)
    L = vec.shape[0]
    cat = jnp.concatenate([cp[0:L], cp[1:1 + L], cp[2:2 + L]], axis=-1)
    return jnp.dot(cat, wcorr, preferred_element_type=f32)


def _relu6(y):
    return jnp.clip(y, 0.0, 6.0)


def _phase4(xv, wph_ref, wtb_ref, wlr_ref, wcn_ref):
    H, W, C = xv.shape
    acc = _conv3raw(xv, wph_ref, pad=_clamp_pad)
    terr = _edge_corr(xv[0], wtb_ref[0])
    berr = _edge_corr(xv[H - 1], wtb_ref[1])
    lerr = _edge_corr(xv[:, 0], wlr_ref[0])
    rerr = _edge_corr(xv[:, W - 1], wlr_ref[1])
    c00 = jnp.dot(xv[0:1, 0], wcn_ref[0], preferred_element_type=f32)
    c0R = jnp.dot(xv[0:1, W - 1], wcn_ref[1], preferred_element_type=f32)
    cB0 = jnp.dot(xv[H - 1:H, 0], wcn_ref[2], preferred_element_type=f32)
    cBR = jnp.dot(xv[H - 1:H, W - 1], wcn_ref[3], preferred_element_type=f32)
    z = jnp.zeros_like(c00)
    terr = jnp.concatenate(
        [terr[0:1] - jnp.concatenate([c00, z], -1), terr[1:W - 1],
         terr[W - 1:] - jnp.concatenate([z, c0R], -1)], 0)
    berr = jnp.concatenate(
        [berr[0:1] - jnp.concatenate([cB0, z], -1), berr[1:W - 1],
         berr[W - 1:] - jnp.concatenate([z, cBR], -1)], 0)
    zw = jnp.zeros((W, 2 * C), f32)
    zh = jnp.zeros((H, C), f32)
    terr_all = jnp.concatenate([terr, zw], -1)
    berr_all = jnp.concatenate([zw, berr], -1)
    lerr_all = jnp.concatenate([lerr[:, :C], zh, lerr[:, C:], zh], -1)
    rerr_all = jnp.concatenate([zh, rerr[:, :C], zh, rerr[:, C:]], -1)
    acc = acc.reshape(H, W, 4 * C)
    acc = jnp.concatenate([acc[0:1] - terr_all[None], acc[1:H - 1],
                           acc[H - 1:] - berr_all[None]], 0)
    acc = jnp.concatenate([acc[:, 0:1] - lerr_all[:, None], acc[:, 1:W - 1],
                           acc[:, W - 1:] - rerr_all[:, None]], 1)
    return acc


def _interleave4(accp, C):
    h, w = accp.shape[0], accp.shape[1]
    a = accp.reshape(h, w, 2, 2, C).transpose(0, 2, 1, 3, 4)
    return a.reshape(2 * h, 2 * w, C)


def _csim_block(spf, xin, pq, w3_sa, sb_sa, w3_ca, sb_ca, watt, bm, wm0, wm1,
                wph, wtb, wlr, wcn, sb_ch, w3_sm, sb_sm):
    H, W, _ = spf.shape
    h, w, C = xin.shape
    sbv = sb_sa[...]
    s_feat = _relu6(_conv3raw(spf, w3_sa) * sbv[0] + sbv[1]).astype(bf16)
    s_pool = jnp.mean(s_feat.astype(f32), axis=0, keepdims=True)
    sbv = sb_ca[...]
    c_feat = _relu6(_conv3raw(xin, w3_ca) * sbv[0] + sbv[1]).astype(bf16)
    c_pool = jnp.mean(c_feat.astype(f32), axis=0, keepdims=True)

    pools = jnp.concatenate([s_pool, c_pool], axis=-1)
    bmv = bm[...]
    atts = jax.nn.sigmoid(
        jnp.dot(pools, watt[...], preferred_element_type=f32) + bmv[2])

    def _hat(v):
        r1 = jnp.concatenate([v[:, 1:], v[:, :1]], -1)
        r2 = jnp.concatenate([v[:, -1:], v[:, :-1]], -1)
        lane = jax.lax.broadcasted_iota(jnp.int32, v.shape, 1)
        vp = jnp.where(lane % 2 == 0, r1, r2)
        nr = jnp.sqrt(v * v + vp * vp)
        return v / jnp.maximum(nr, 1e-12)

    sh = _hat(atts[:, :C])
    chh = _hat(atts[:, C:])
    se1 = jnp.dot(sh, pq[0][...], preferred_element_type=f32)
    se2 = jnp.dot(sh, pq[1][...], preferred_element_type=f32)
    ce1 = jnp.dot(chh, pq[2][...], preferred_element_type=f32)
    ce2 = jnp.dot(chh, pq[3][...], preferred_element_type=f32)
    aff = se1 * ce1 + se2 * ce2
    h1 = jnp.maximum(
        jnp.dot(aff, wm0[...], preferred_element_type=f32) + bmv[0], 0.0)
    h2 = jnp.maximum(
        jnp.dot(h1, wm1[...], preferred_element_type=f32) + bmv[1], 0.0)
    re = jax.nn.sigmoid(atts + bmv[3] * h2)
    re_s = re[:, :C].astype(bf16).reshape(1, 1, C)
    re_c = re[:, C:].astype(bf16).reshape(1, 1, C)

    accp = _phase4(c_feat.reshape(h, w, C) * re_c, wph, wtb, wlr, wcn)
    sbv = sb_ch[...]
    ch = _relu6(_interleave4(accp, C).reshape(H * W, C) * sbv[0] + sbv[1])
    xin2 = s_feat.reshape(H, W, C) * re_s + ch.astype(bf16).reshape(H, W, C)
    sbv = sb_sm[...]
    y = _conv3raw(xin2, w3_sm) * sbv[0] + sbv[1]
    y = y * y + y
    return y.astype(bf16).reshape(H, W, C)


def _decoder_kernel(*refs, Hr, Wr):
    it = iter(refs)
    r4 = next(it)[0]
    r3 = next(it)[0]
    r2 = next(it)[0]
    r1 = next(it)[0]
    wpc1, sbpc1 = next(it), next(it)
    pq = [next(it) for _ in range(4)]
    p3refs = [next(it) for _ in range(15)]
    p2refs = [next(it) for _ in range(15)]
    (wcomp, wphf, wtbf, wlrf, wcnf, sbpost, wca1, wca2) = (
        next(it) for _ in range(8))
    (dww, sbpa, wproj, sbproj, wsc, sbsc, wph, sbseg, wtb, wlr, wcn,
     wcls) = (next(it) for _ in range(12))
    o_ref = next(it)

    H4, W4, C4 = r4.shape
    sbv = sbpc1[...]
    y = jnp.dot(r4.reshape(H4 * W4, C4), wpc1[...],
                preferred_element_type=f32)
    y = y * sbv[0] + sbv[1]
    y = y * y + y
    x8 = y.astype(bf16).reshape(H4, W4, y.shape[-1])

    x16 = _csim_block(r3, x8, pq, *p3refs)
    x32 = _csim_block(r2, x16, pq, *p2refs)

    C = x32.shape[-1]
    acc1 = _conv3raw(r1, wcomp)
    accp = _phase4(x32, wphf, wtbf, wlrf, wcnf)
    sbv = sbpost[...]
    y = (acc1 + _interleave4(accp, C).reshape(Hr * Wr, C)) * sbv[0] + sbv[1]
    xpf = _relu6(y).astype(bf16)
    pooled = jnp.mean(xpf.astype(f32), axis=0, keepdims=True)
    hca = _relu6(jnp.dot(pooled, wca1[...], preferred_element_type=f32))
    cav = jax.nn.sigmoid(jnp.dot(hca, wca2[...], preferred_element_type=f32))

    xpv = xpf.reshape(Hr, Wr, C)
    dwacc = _conv3raw(xpv, dww)
    y = jax.nn.sigmoid(dwacc + sbpa[1])
    gated = ((y + cav) * xpf.astype(f32)).astype(bf16)

    yp = _conv3raw(gated.reshape(Hr, Wr, C), wproj)
    sbv = sbproj[...]
    yp = yp * sbv[0] + sbv[1]
    sc = jnp.dot(xpf, wsc[...], preferred_element_type=f32)
    sbv = sbsc[...]
    x64 = _relu6(yp + sc * sbv[0] + sbv[1]).astype(bf16)

    acc = _phase4(x64.reshape(Hr, Wr, C), wph, wtb, wlr, wcn)
    sbv = sbseg[...]
    y = acc.reshape(Hr * Wr, 4 * C) * sbv[0] + sbv[1]
    y = _relu6(y).astype(bf16)
    o_ref[0] = jnp.dot(y, wcls[...], preferred_element_type=f32)


def _bn_fold(g, b, m, v):
    s = g / jnp.sqrt(v + 1e-5)
    return s, b - m * s


def _sb_pack(c, scale=None, bias=None):
    sb = jnp.zeros((8, c), f32)
    sb = sb.at[0].set(jnp.ones((c,), f32) if scale is None else scale)
    if bias is not None:
        sb = sb.at[1].set(bias)
    return sb


def _w3(w):
    return w.reshape(3, 3 * w.shape[2], w.shape[3]).astype(bf16)


_M0 = jnp.array([[0.75, 0.25, 0.0], [0.25, 0.75, 0.75], [0.0, 0.0, 0.25]],
                f32)
_M1 = jnp.array([[0.25, 0.0, 0.0], [0.75, 0.75, 0.25], [0.0, 0.25, 0.75]],
                f32)


def _phase_weights(K):
    Mp = (_M0, _M1)
    ci, co = K.shape[2], K.shape[3]
    wph = jnp.concatenate([
        jnp.einsum("ad,be,deio->abio", Mp[py], Mp[px], K).reshape(
            3, 3 * ci, co)
        for py in (0, 1) for px in (0, 1)], axis=-1).astype(bf16)
    wtb = jnp.stack([
        jnp.concatenate([jnp.einsum("be,eio->bio", Mpx, K[d]).reshape(
            3 * ci, co) for Mpx in Mp], axis=-1)
        for d in (0, 2)]).astype(bf16)
    wlr = jnp.stack([
        jnp.concatenate([jnp.einsum("ad,dio->aio", Mpy, K[:, d]).reshape(
            3 * ci, co) for Mpy in Mp], axis=-1)
        for d in (0, 2)]).astype(bf16)
    wcn = jnp.stack([K[0, 0], K[0, 2], K[2, 0], K[2, 2]]).astype(bf16)
    return wph, wtb, wlr, wcn


_PPAR = pltpu.CompilerParams(dimension_semantics=("parallel",))


def _full(shape):
    nd = len(shape)
    return pl.BlockSpec(shape, lambda n, _nd=nd: (0,) * _nd)


def _img(shape):
    nd = len(shape) - 1
    return pl.BlockSpec((1,) + shape[1:],
                        lambda n, _nd=nd: (n,) + (0,) * _nd)


def _csim_operands(dc, g1, g2, sp0_w, sp0_b, sp1_w, sp1_b, cx0_w, cx0_b,
                   cx1_w, cx1_b, sa_cbr, sa_c11, ca_cbr, ca_c11, ch, sm):
    ss, st = _bn_fold(*sa_cbr[1:])
    cs, ct = _bn_fold(*ca_cbr[1:])
    s1, t1 = _bn_fold(*sa_c11[1:])
    s2, t2 = _bn_fold(*ca_c11[1:])
    watt = jnp.zeros((2 * dc, 2 * dc), f32)
    watt = watt.at[:dc, :dc].set(sa_c11[0].reshape(dc, dc) * s1[None, :])
    watt = watt.at[dc:, dc:].set(ca_c11[0].reshape(dc, dc) * s2[None, :])
    bm = jnp.zeros((8, 2 * dc), f32)
    bm = bm.at[0].set(jnp.concatenate([sp0_b, cx0_b]))
    bm = bm.at[1].set(jnp.concatenate([sp1_b, cx1_b]))
    bm = bm.at[2].set(jnp.concatenate([t1, t2]))
    bm = bm.at[3].set(jnp.concatenate([g1 * jnp.ones((dc,), f32),
                                       g2 * jnp.ones((dc,), f32)]))
    wm0 = jnp.concatenate([sp0_w, cx0_w], axis=1).astype(f32)
    wm1 = jnp.zeros((2 * dc, 2 * dc), f32)
    wm1 = wm1.at[:dc, :dc].set(sp1_w)
    wm1 = wm1.at[dc:, dc:].set(cx1_w)
    wph, wtb, wlr, wcn = _phase_weights(ch[0].astype(f32))
    chs, cht = _bn_fold(*ch[1:])
    sms, smt = _bn_fold(*sm[1:])
    return [_w3(sa_cbr[0]), _sb_pack(dc, ss, st),
            _w3(ca_cbr[0]), _sb_pack(dc, cs, ct),
            watt, bm, wm0, wm1, wph, wtb, wlr, wcn,
            _sb_pack(dc, chs, cht), _w3(sm[0]), _sb_pack(dc, sms, smt)]


def kernel(res1, res2, res3, res4, pc1_w, pc1_g, pc1_b, pc1_m, pc1_v, p3_g1, p3_g2, p3_sp0_w, p3_sp0_b, p3_sp1_w, p3_sp1_b, p3_cx0_w, p3_cx0_b, p3_cx1_w, p3_cx1_b, p3_sa_cbr_w, p3_sa_cbr_g, p3_sa_cbr_b, p3_sa_cbr_m, p3_sa_cbr_v, p3_sa_c11_w, p3_sa_c11_g, p3_sa_c11_b, p3_sa_c11_m, p3_sa_c11_v, p3_ca_cbr_w, p3_ca_cbr_g, p3_ca_cbr_b, p3_ca_cbr_m, p3_ca_cbr_v, p3_ca_c11_w, p3_ca_c11_g, p3_ca_c11_b, p3_ca_c11_m, p3_ca_c11_v, p3_ch_w, p3_ch_g, p3_ch_b, p3_ch_m, p3_ch_v, p3_sm_w, p3_sm_g, p3_sm_b, p3_sm_m, p3_sm_v, p2_g1, p2_g2, p2_sp0_w, p2_sp0_b, p2_sp1_w, p2_sp1_b, p2_cx0_w, p2_cx0_b, p2_cx1_w, p2_cx1_b, p2_sa_cbr_w, p2_sa_cbr_g, p2_sa_cbr_b, p2_sa_cbr_m, p2_sa_cbr_v, p2_sa_c11_w, p2_sa_c11_g, p2_sa_c11_b, p2_sa_c11_m, p2_sa_c11_v, p2_ca_cbr_w, p2_ca_cbr_g, p2_ca_cbr_b, p2_ca_cbr_m, p2_ca_cbr_v, p2_ca_c11_w, p2_ca_c11_g, p2_ca_c11_b, p2_ca_c11_m, p2_ca_c11_v, p2_ch_w, p2_ch_g, p2_ch_b, p2_ch_m, p2_ch_v, p2_sm_w, p2_sm_g, p2_sm_b, p2_sm_m, p2_sm_v, p1_pre_w, p1_weights, p1_post_w, p1_post_g, p1_post_b, p1_post_m, p1_post_v, p1_pa_w, p1_pa_b, p1_ca1_w, p1_ca2_w, p1_sc_w, p1_sc_g, p1_sc_b, p1_sc_m, p1_sc_v, p1_proj_w, p1_proj_g, p1_proj_b, p1_proj_m, p1_proj_v, seg1_w, seg1_g, seg1_b, seg1_m, seg1_v, seg2_w):
    t = lambda a: jnp.transpose(a, (0, 2, 3, 1)).astype(bf16)
    r1, r2, r3, r4 = t(res1), t(res2), t(res3), t(res4)
    N = r4.shape[0]
    dc = 32
    nc = seg2_w.shape[-1]
    Hr, Wr = r1.shape[1], r1.shape[2]

    operands = [r4, r3, r2, r1]
    specs = [_img(r4.shape), _img(r3.shape), _img(r2.shape), _img(r1.shape)]

    def add(arr):
        operands.append(arr)
        specs.append(_full(arr.shape))

    s, b = _bn_fold(pc1_g, pc1_b, pc1_m, pc1_v)
    add(pc1_w.reshape(pc1_w.shape[2], dc).astype(bf16))
    add(_sb_pack(dc, s, b))

    qi = jnp.arange(256)
    li = jnp.arange(2 * dc)[:, None]
    add((li == 2 * (qi // 16)[None, :]).astype(f32)[:dc])
    add((li == 2 * (qi // 16)[None, :] + 1).astype(f32)[:dc])
    add((li == 2 * (qi % 16)[None, :]).astype(f32)[:dc])
    add((li == 2 * (qi % 16)[None, :] + 1).astype(f32)[:dc])

    for ops in (
            _csim_operands(dc, p3_g1, p3_g2, p3_sp0_w, p3_sp0_b, p3_sp1_w,
                           p3_sp1_b, p3_cx0_w, p3_cx0_b, p3_cx1_w, p3_cx1_b,
                           (p3_sa_cbr_w, p3_sa_cbr_g, p3_sa_cbr_b,
                            p3_sa_cbr_m, p3_sa_cbr_v),
                           (p3_sa_c11_w, p3_sa_c11_g, p3_sa_c11_b,
                            p3_sa_c11_m, p3_sa_c11_v),
                           (p3_ca_cbr_w, p3_ca_cbr_g, p3_ca_cbr_b,
                            p3_ca_cbr_m, p3_ca_cbr_v),
                           (p3_ca_c11_w, p3_ca_c11_g, p3_ca_c11_b,
                            p3_ca_c11_m, p3_ca_c11_v),
                           (p3_ch_w, p3_ch_g, p3_ch_b, p3_ch_m, p3_ch_v),
                           (p3_sm_w, p3_sm_g, p3_sm_b, p3_sm_m, p3_sm_v)),
            _csim_operands(dc, p2_g1, p2_g2, p2_sp0_w, p2_sp0_b, p2_sp1_w,
                           p2_sp1_b, p2_cx0_w, p2_cx0_b, p2_cx1_w, p2_cx1_b,
                           (p2_sa_cbr_w, p2_sa_cbr_g, p2_sa_cbr_b,
                            p2_sa_cbr_m, p2_sa_cbr_v),
                           (p2_sa_c11_w, p2_sa_c11_g, p2_sa_c11_b,
                            p2_sa_c11_m, p2_sa_c11_v),
                           (p2_ca_cbr_w, p2_ca_cbr_g, p2_ca_cbr_b,
                            p2_ca_cbr_m, p2_ca_cbr_v),
                           (p2_ca_c11_w, p2_ca_c11_g, p2_ca_c11_b,
                            p2_ca_c11_m, p2_ca_c11_v),
                           (p2_ch_w, p2_ch_g, p2_ch_b, p2_ch_m, p2_ch_v),
                           (p2_sm_w, p2_sm_g, p2_sm_b, p2_sm_m, p2_sm_v))):
        for o in ops:
            add(o)

    Ci = r1.shape[-1]
    w_pos = jnp.maximum(p1_weights, 0.0)
    fw = w_pos / (jnp.sum(w_pos) + 1e-8)
    Kpost = p1_post_w.astype(f32)
    add((jnp.einsum("ij,dejo->deio", p1_pre_w.reshape(Ci, dc).astype(f32),
                    Kpost) * fw[0]).reshape(3, 3 * Ci, dc).astype(bf16))
    for o in _phase_weights(Kpost * fw[1]):
        add(o)
    ps, pt = _bn_fold(p1_post_g, p1_post_b, p1_post_m, p1_post_v)
    add(_sb_pack(dc, ps, pt))
    wca1 = jnp.zeros((dc, 8), f32).at[:, :dc // 16].set(
        p1_ca1_w.reshape(dc, dc // 16))
    wca2 = jnp.zeros((8, dc), f32).at[:dc // 16, :].set(
        p1_ca2_w.reshape(dc // 16, dc))
    add(wca1)
    add(wca2)

    idx = jnp.arange(dc)
    dww = jnp.zeros((3, 3 * dc, dc), f32)
    for dy in range(3):
        for dx in range(3):
            dww = dww.at[dy, dx * dc + idx, idx].set(p1_pa_w[dy, dx])
    add(dww.astype(bf16))
    add(_sb_pack(dc, None, p1_pa_b))
    pjs, pjt = _bn_fold(p1_proj_g, p1_proj_b, p1_proj_m, p1_proj_v)
    add(_w3(p1_proj_w))
    add(_sb_pack(dc, pjs, pjt))
    scs, sct = _bn_fold(p1_sc_g, p1_sc_b, p1_sc_m, p1_sc_v)
    add(p1_sc_w.reshape(dc, dc).astype(bf16))
    add(_sb_pack(dc, scs, sct))
    wph, wtb, wlr, wcn = _phase_weights(seg1_w.astype(f32))
    sgs, sgt = _bn_fold(seg1_g, seg1_b, seg1_m, seg1_v)
    add(wph)
    add(_sb_pack(4 * dc, jnp.tile(sgs, 4), jnp.tile(sgt, 4)))
    add(wtb)
    add(wlr)
    add(wcn)
    w2c = seg2_w.reshape(dc, nc).astype(f32)
    wcls = jnp.zeros((4 * dc, 4 * nc), f32)
    for p in range(4):
        wcls = wcls.at[p * dc:(p + 1) * dc, p * nc:(p + 1) * nc].set(w2c)
    add(wcls.astype(bf16))

    out = pl.pallas_call(
        functools.partial(_decoder_kernel, Hr=Hr, Wr=Wr),
        out_shape=jax.ShapeDtypeStruct((N, Hr * Wr, 4 * nc), f32),
        grid=(N,),
        in_specs=specs,
        out_specs=pl.BlockSpec((1, Hr * Wr, 4 * nc), lambda n: (n, 0, 0)),
        compiler_params=_PPAR,
    )(*operands)

    out = out.reshape(N, Hr, Wr, 2, 2, nc)
    return jnp.transpose(out, (0, 5, 1, 3, 2, 4)).reshape(
        N, nc, 2 * Hr, 2 * Wr)
